# Initial kernel scaffold; baseline (speedup 1.0000x reference)
import jax
import jax.numpy as jnp
from jax.experimental import pallas as pl


def kernel(node_feats, edge_index, W1, b1, W2, b2):
    raise NotImplementedError("write your pallas kernel here")


# speedup vs baseline: 5.4212x; 5.4212x over previous
import functools

import jax
import jax.numpy as jnp
from jax import lax
from jax.experimental import pallas as pl
from jax.experimental.pallas import tpu as pltpu
from jax.experimental.pallas import tpu_sc as plsc

N_NODES = 10000
D = 128
E = 320000

NC = 2
NS = 16
NW = NC * NS
EPW = E // NW
K = 80
NCHUNK = EPW // K
CP = 200
NBLK = N_NODES // CP


def _relu_linear_body(x_ref, w_ref, b_ref, o_ref):
    acc = jnp.dot(x_ref[...], w_ref[...], preferred_element_type=jnp.float32)
    o_ref[...] = jnp.maximum(acc + b_ref[...], 0.0)


def _relu_linear(x, w1t, b1):
    nb = 10
    rb = N_NODES // nb
    return pl.pallas_call(
        _relu_linear_body,
        grid=(nb,),
        in_specs=[
            pl.BlockSpec((rb, D), lambda i: (i, 0)),
            pl.BlockSpec((D, D), lambda i: (0, 0)),
            pl.BlockSpec((1, D), lambda i: (0, 0)),
        ],
        out_specs=pl.BlockSpec((rb, D), lambda i: (i, 0)),
        out_shape=jax.ShapeDtypeStruct((N_NODES, D), jnp.float32),
    )(x, w1t, b1)


def _combine_body(p_ref, w_ref, b_ref, o_ref):
    s = p_ref[0] + p_ref[1]
    acc = jnp.dot(s, w_ref[...], preferred_element_type=jnp.float32)
    o_ref[...] = acc + b_ref[...]


def _combine(partials, w2t, b2):
    nb = 10
    rb = N_NODES // nb
    return pl.pallas_call(
        _combine_body,
        grid=(nb,),
        in_specs=[
            pl.BlockSpec((NC, rb, D), lambda i: (0, i, 0)),
            pl.BlockSpec((D, D), lambda i: (0, 0)),
            pl.BlockSpec((1, D), lambda i: (0, 0)),
        ],
        out_specs=pl.BlockSpec((rb, D), lambda i: (i, 0)),
        out_shape=jax.ShapeDtypeStruct((N_NODES, D), jnp.float32),
    )(partials, w2t, b2)


def _scatter_body(h_hbm, src_hbm, dst_hbm, out_hbm, acc_sh, src_v, dst_v,
                  rows_v, stage_v, sem):
    cid = lax.axis_index("c")
    sid = lax.axis_index("s")
    wid = sid * NC + cid

    def _zero_row(i, _):
        for c in range(D // 16):
            stage_v[i, pl.ds(c * 16, 16)] = jnp.zeros((16,), jnp.float32)
        return 0
    lax.fori_loop(0, CP, _zero_row, 0)
    for j in range((NBLK + NS - 1) // NS):
        blk = j * NS + sid
        @pl.when(blk < NBLK)
        def _():
            pltpu.sync_copy(stage_v, acc_sh.at[pl.ds(blk * CP, CP)])
    plsc.subcore_barrier()

    ebase = wid * EPW

    def _chunk(ci, _):
        off = ebase + ci * K
        pltpu.sync_copy(src_hbm.at[pl.ds(off, K)], src_v)
        pltpu.sync_copy(dst_hbm.at[pl.ds(off, K)], dst_v)
        pltpu.async_copy(h_hbm.at[src_v], rows_v, sem).wait()
        pltpu.sync_copy(rows_v, acc_sh.at[dst_v], add=True)
        return 0
    lax.fori_loop(0, NCHUNK, _chunk, 0)
    plsc.subcore_barrier()

    for j in range((NBLK + NS - 1) // NS):
        blk = j * NS + sid
        @pl.when(blk < NBLK)
        def _():
            r0 = blk * CP
            pltpu.sync_copy(acc_sh.at[pl.ds(r0, CP)], stage_v)
            pltpu.sync_copy(stage_v, out_hbm.at[cid, pl.ds(r0, CP)])


def _edge_scatter(h, src, dst):
    mesh = plsc.VectorSubcoreMesh(core_axis_name="c", subcore_axis_name="s")
    kern = pl.kernel(
        _scatter_body,
        out_type=jax.ShapeDtypeStruct((NC, N_NODES, D), jnp.float32),
        mesh=mesh,
        scratch_types=[
            pltpu.VMEM_SHARED((N_NODES, D), jnp.float32),
            pltpu.VMEM((K,), jnp.int32),
            pltpu.VMEM((K,), jnp.int32),
            pltpu.VMEM((K, D), jnp.float32),
            pltpu.VMEM((CP, D), jnp.float32),
            pltpu.SemaphoreType.DMA,
        ],
    )
    return kern(h, src, dst)


def kernel(node_feats, edge_index, W1, b1, W2, b2):
    w1t = W1.T
    w2t = W2.T
    b1r = b1.reshape(1, D)
    b2r = b2.reshape(1, D)
    h = _relu_linear(node_feats, w1t, b1r)
    partials = _edge_scatter(h, edge_index[0], edge_index[1])
    return _combine(partials, w2t, b2r)


# speedup vs baseline: 10.9404x; 2.0181x over previous
import functools

import jax
import jax.numpy as jnp
from jax import lax
from jax.experimental import pallas as pl
from jax.experimental.pallas import tpu as pltpu
from jax.experimental.pallas import tpu_sc as plsc

N_NODES = 10000
D = 128
E = 320000

NC = 2
NS = 16
NW = NC * NS
EPW = E // NW
K = 128
NCHUNK = EPW // K
KTAIL = EPW - NCHUNK * K
CP = 80
NBLK = N_NODES // CP


def _relu_linear_body(x_ref, w_ref, b_ref, o_ref):
    acc = jnp.dot(x_ref[...], w_ref[...], preferred_element_type=jnp.float32)
    o_ref[...] = jnp.maximum(acc + b_ref[...], 0.0)


def _relu_linear(x, w1t, b1):
    nb = 10
    rb = N_NODES // nb
    return pl.pallas_call(
        _relu_linear_body,
        grid=(nb,),
        in_specs=[
            pl.BlockSpec((rb, D), lambda i: (i, 0)),
            pl.BlockSpec((D, D), lambda i: (0, 0)),
            pl.BlockSpec((1, D), lambda i: (0, 0)),
        ],
        out_specs=pl.BlockSpec((rb, D), lambda i: (i, 0)),
        out_shape=jax.ShapeDtypeStruct((N_NODES, D), jnp.float32),
    )(x, w1t, b1)


def _combine_body(p_ref, w_ref, b_ref, o_ref):
    s = p_ref[0] + p_ref[1]
    acc = jnp.dot(s, w_ref[...], preferred_element_type=jnp.float32)
    o_ref[...] = acc + b_ref[...]


def _combine(partials, w2t, b2):
    nb = 10
    rb = N_NODES // nb
    return pl.pallas_call(
        _combine_body,
        grid=(nb,),
        in_specs=[
            pl.BlockSpec((NC, rb, D), lambda i: (0, i, 0)),
            pl.BlockSpec((D, D), lambda i: (0, 0)),
            pl.BlockSpec((1, D), lambda i: (0, 0)),
        ],
        out_specs=pl.BlockSpec((rb, D), lambda i: (i, 0)),
        out_shape=jax.ShapeDtypeStruct((N_NODES, D), jnp.float32),
    )(partials, w2t, b2)


def _scatter_body(h_hbm, src_hbm, dst_hbm, out_hbm, acc_sh, src_v0, src_v1,
                  dst_v0, dst_v1, rows_v, stage_v, tsrc_v, tdst_v, trows_v,
                  isem0, isem1, gsem0, gsem1):
    src_v = [src_v0, src_v1]
    dst_v = [dst_v0, dst_v1]
    isem = [isem0, isem1]
    gsem = [gsem0, gsem1]
    cid = lax.axis_index("c")
    sid = lax.axis_index("s")
    wid = sid * NC + cid

    def _zero_row(i, _):
        for c in range(D // 16):
            stage_v[i, pl.ds(c * 16, 16)] = jnp.zeros((16,), jnp.float32)
        return 0
    lax.fori_loop(0, CP, _zero_row, 0)
    for j in range((NBLK + NS - 1) // NS):
        blk = j * NS + sid
        @pl.when(blk < NBLK)
        def _():
            pltpu.sync_copy(stage_v, acc_sh.at[pl.ds(blk * CP, CP)])
    plsc.subcore_barrier()

    ebase = wid * EPW

    def _start_idx(ci, b):
        off = ebase + ci * K
        pltpu.async_copy(src_hbm.at[pl.ds(off, K)], src_v[b], isem[b])
        pltpu.async_copy(dst_hbm.at[pl.ds(off, K)], dst_v[b], isem[b])

    def _wait_idx(b):
        pltpu.make_async_copy(src_hbm.at[pl.ds(0, K)], src_v[b],
                              isem[b]).wait()
        pltpu.make_async_copy(dst_hbm.at[pl.ds(0, K)], dst_v[b],
                              isem[b]).wait()

    def _start_gather(b):
        pltpu.async_copy(h_hbm.at[src_v[b]], rows_v.at[b], gsem[b])

    def _wait_gather(b):
        pltpu.make_async_copy(h_hbm.at[pl.ds(0, K)], rows_v.at[b],
                              gsem[b]).wait()

    _start_idx(0, 0)
    _start_idx(1, 1)
    _wait_idx(0)
    _start_gather(0)

    def _chunk(half, _):
        for b in range(2):
            ci = half * 2 + b
            nxt = 1 - b
            _wait_gather(b)

            @pl.when(ci + 1 < NCHUNK)
            def _():
                _wait_idx(nxt)
                _start_gather(nxt)
            pltpu.sync_copy(rows_v.at[b], acc_sh.at[dst_v[b]], add=True)

            @pl.when(ci + 2 < NCHUNK)
            def _():
                _start_idx(ci + 2, b)
        return 0
    lax.fori_loop(0, NCHUNK // 2, _chunk, 0)

    if KTAIL:
        toff = ebase + NCHUNK * K
        pltpu.sync_copy(src_hbm.at[pl.ds(toff, KTAIL)], tsrc_v)
        pltpu.sync_copy(dst_hbm.at[pl.ds(toff, KTAIL)], tdst_v)
        pltpu.async_copy(h_hbm.at[tsrc_v], trows_v, gsem[0]).wait()
        pltpu.sync_copy(trows_v, acc_sh.at[tdst_v], add=True)
    plsc.subcore_barrier()

    for j in range((NBLK + NS - 1) // NS):
        blk = j * NS + sid
        @pl.when(blk < NBLK)
        def _():
            r0 = blk * CP
            pltpu.sync_copy(acc_sh.at[pl.ds(r0, CP)], stage_v)
            pltpu.sync_copy(stage_v, out_hbm.at[cid, pl.ds(r0, CP)])


def _edge_scatter(h, src, dst):
    mesh = plsc.VectorSubcoreMesh(core_axis_name="c", subcore_axis_name="s")
    kern = pl.kernel(
        _scatter_body,
        out_type=jax.ShapeDtypeStruct((NC, N_NODES, D), jnp.float32),
        mesh=mesh,
        scratch_types=[
            pltpu.VMEM_SHARED((N_NODES, D), jnp.float32),
            pltpu.VMEM((K,), jnp.int32),
            pltpu.VMEM((K,), jnp.int32),
            pltpu.VMEM((K,), jnp.int32),
            pltpu.VMEM((K,), jnp.int32),
            pltpu.VMEM((2, K, D), jnp.float32),
            pltpu.VMEM((CP, D), jnp.float32),
            pltpu.VMEM((KTAIL,), jnp.int32),
            pltpu.VMEM((KTAIL,), jnp.int32),
            pltpu.VMEM((KTAIL, D), jnp.float32),
            pltpu.SemaphoreType.DMA,
            pltpu.SemaphoreType.DMA,
            pltpu.SemaphoreType.DMA,
            pltpu.SemaphoreType.DMA,
        ],
    )
    return kern(h, src, dst)


def kernel(node_feats, edge_index, W1, b1, W2, b2):
    w1t = W1.T
    w2t = W2.T
    b1r = b1.reshape(1, D)
    b2r = b2.reshape(1, D)
    h = _relu_linear(node_feats, w1t, b1r)
    partials = _edge_scatter(h, edge_index[0], edge_index[1])
    return _combine(partials, w2t, b2r)
